# baseline (speedup 1.0000x reference)
import functools

import jax
import jax.numpy as jnp
from jax import lax
from jax.experimental import pallas as pl
from jax.experimental.pallas import tpu as pltpu
from jax.experimental.pallas import tpu_sc as plsc

N_ROWS = 100000
N_COLS = 256
N_IDX = N_ROWS * N_COLS
NC, NS = 2, 16
NW = NC * NS
PER_W = N_IDX // NW
CHUNK = 10000
NCH = PER_W // CHUNK
M_PAD = 100352
NPASS = 14
HALF = M_PAD // NPASS
HSLICE = HALF // NS
ROW_BLK = 1000
NG = N_ROWS // ROW_BLK


def _sc_mark_hits(idx_hbm, out_hbm, mask_v, buf0, buf1, shared, sem0, sem1):
    c = lax.axis_index("c")
    s = lax.axis_index("s")
    wid = s * NC + c
    base = wid * PER_W

    zeros16 = jnp.zeros((16,), jnp.float32)
    ones16 = jnp.ones((16,), jnp.float32)

    @plsc.parallel_loop(0, M_PAD // 16, 1, unroll=8)
    def zero_body(i):
        mask_v[pl.ds(i * 16, 16)] = zeros16

    bufs = (buf0, buf1)
    sems = (sem0, sem1)
    descs = [None] * NCH
    for ci in range(2):
        descs[ci] = pltpu.async_copy(
            idx_hbm.at[pl.ds(base + ci * CHUNK, CHUNK)], bufs[ci], sems[ci])

    for ci in range(NCH):
        buf = bufs[ci % 2]
        descs[ci].wait()

        @plsc.parallel_loop(0, CHUNK // 16, 1, unroll=8)
        def scat_body(j, buf=buf):
            iv = buf[pl.ds(j * 16, 16)]
            plsc.store_scatter(mask_v, [iv], ones16)

        if ci + 2 < NCH:
            descs[ci + 2] = pltpu.async_copy(
                idx_hbm.at[pl.ds(base + (ci + 2) * CHUNK, CHUNK)],
                buf, sems[ci % 2])

    hoff = s * HSLICE

    def pass_body(p, carry):
        plsc.subcore_barrier()
        pltpu.sync_copy(mask_v.at[pl.ds(p * HALF, HALF)],
                        shared.at[pl.ds(s * HALF, HALF)])
        plsc.subcore_barrier()

        pltpu.sync_copy(shared.at[pl.ds(hoff, HSLICE)],
                        mask_v.at[pl.ds(0, HSLICE)])

        def slot_body(j, carry2):
            pltpu.sync_copy(shared.at[pl.ds(j * HALF + hoff, HSLICE)],
                            mask_v.at[pl.ds(HSLICE, HSLICE)])

            @plsc.parallel_loop(0, HSLICE // 16, 1, unroll=4)
            def add_body(i):
                a = mask_v[pl.ds(i * 16, 16)]
                b = mask_v[pl.ds(HSLICE + i * 16, 16)]
                mask_v[pl.ds(i * 16, 16)] = a + b

            return carry2

        lax.fori_loop(1, NS, slot_body, None)
        pltpu.sync_copy(mask_v.at[pl.ds(0, HSLICE)],
                        out_hbm.at[pl.ds(c * M_PAD + p * HALF + hoff, HSLICE)])
        return carry

    lax.fori_loop(0, NPASS, pass_body, None)


_sc_mark_hits_call = functools.partial(
    pl.kernel,
    mesh=plsc.VectorSubcoreMesh(core_axis_name="c", subcore_axis_name="s"),
    out_type=jax.ShapeDtypeStruct((NC * M_PAD,), jnp.float32),
    scratch_types=[
        pltpu.VMEM((M_PAD,), jnp.float32),
        pltpu.VMEM((CHUNK,), jnp.int32),
        pltpu.VMEM((CHUNK,), jnp.int32),
        pltpu.VMEM_SHARED((NS * HALF,), jnp.float32),
        pltpu.SemaphoreType.DMA,
        pltpu.SemaphoreType.DMA,
    ],
    compiler_params=pltpu.CompilerParams(needs_layout_passes=False),
)(_sc_mark_hits)


def _tc_rowsum(t_ref, out_ref):
    tbt = t_ref[...].T
    r_t = jnp.sum(tbt, axis=0, keepdims=True)
    out_ref[...] = r_t.reshape(1, 1, ROW_BLK)


def _tc_rowsum_call(t):
    return pl.pallas_call(
        _tc_rowsum,
        grid=(NG,),
        in_specs=[
            pl.BlockSpec((ROW_BLK, N_COLS), lambda i: (i, 0)),
        ],
        out_specs=pl.BlockSpec((1, 1, ROW_BLK), lambda i: (i, 0, 0)),
        out_shape=jax.ShapeDtypeStruct((NG, 1, ROW_BLK), jnp.float32),
    )(t)


def _tc_final(hm_ref, r_ref, v_ref, out_ref):
    m = hm_ref[0] + hm_ref[1]
    r = r_ref[...]
    miss = jnp.where(m > 0.0, 0.0, r)
    nmiss = jnp.where(m > 0.0, 0.0, 1.0)
    s = jnp.sum(miss)
    n = jnp.sum(nmiss)
    v = v_ref[...]
    out_ref[...] = (s + (N_ROWS - n) * float(N_COLS) * v) / float(N_IDX)


def _tc_final_call(hm, rsum, val2d):
    return pl.pallas_call(
        _tc_final,
        grid=(1,),
        in_specs=[
            pl.BlockSpec((NC, NG, 1, ROW_BLK), lambda i: (0, 0, 0, 0)),
            pl.BlockSpec((NG, 1, ROW_BLK), lambda i: (0, 0, 0)),
            pl.BlockSpec((1, 1), lambda i: (0, 0)),
        ],
        out_specs=pl.BlockSpec((1, 1), lambda i: (0, 0)),
        out_shape=jax.ShapeDtypeStruct((1, 1), jnp.float32),
    )(hm, rsum, val2d)


def kernel(t):
    assert t.shape == (N_ROWS, N_COLS)
    k1, k2 = jax.random.split(jax.random.key(1))
    index = jax.random.randint(k1, (N_IDX,), 0, t.shape[0], dtype=jnp.int32)
    val = jax.random.normal(k2, (1,), dtype=t.dtype)

    hits = _sc_mark_hits_call(index)
    hm = hits.reshape(NC, M_PAD)[:, :N_ROWS].reshape(NC, NG, 1, ROW_BLK)
    rsum = _tc_rowsum_call(t)
    out = _tc_final_call(hm, rsum, val.reshape(1, 1))
    return out[0, 0]


# speedup vs baseline: 1.0155x; 1.0155x over previous
import functools

import jax
import jax.numpy as jnp
from jax import lax
from jax.experimental import pallas as pl
from jax.experimental.pallas import tpu as pltpu
from jax.experimental.pallas import tpu_sc as plsc
from jax._src.random.threefry2x32 import threefry2x32_p

N_ROWS = 100000
N_COLS = 256
N_IDX = N_ROWS * N_COLS
NC, NS = 2, 16
NW = NC * NS
NCHK = 4
CH_N = N_IDX // NCHK
PER_C = CH_N // NW
CHUNK = 10000
NCH_C = PER_C // CHUNK
M_PAD = 100352
NPASS = 14
HALF = M_PAD // NPASS
HSLICE = HALF // NS
ROW_BLK = 1000
NG = N_ROWS // ROW_BLK
SPAN = 100000


def _sc_scatter(init, combine, idx_hbm, *refs):
    if init:
        (out_hbm, mask_v, buf0, buf1, sem0, sem1) = refs
        state_in = shared = None
    elif combine:
        (state_in, out_hbm, mask_v, buf0, buf1, shared, sem0, sem1) = refs
    else:
        (state_in, out_hbm, mask_v, buf0, buf1, sem0, sem1) = refs
        shared = None

    c = lax.axis_index("c")
    s = lax.axis_index("s")
    wid = s * NC + c
    base = wid * PER_C

    zeros16 = jnp.zeros((16,), jnp.float32)
    ones16 = jnp.ones((16,), jnp.float32)

    bufs = (buf0, buf1)
    sems = (sem0, sem1)
    descs = [None] * NCH_C
    for ci in range(2):
        descs[ci] = pltpu.async_copy(
            idx_hbm.at[pl.ds(base + ci * CHUNK, CHUNK)], bufs[ci], sems[ci])

    if init:
        @plsc.parallel_loop(0, M_PAD // 16, 1, unroll=8)
        def zero_body(i):
            mask_v[pl.ds(i * 16, 16)] = zeros16
    else:
        pltpu.sync_copy(state_in.at[pl.ds(wid * M_PAD, M_PAD)], mask_v)

    for ci in range(NCH_C):
        buf = bufs[ci % 2]
        descs[ci].wait()

        @plsc.parallel_loop(0, CHUNK // 16, 1, unroll=8)
        def scat_body(j, buf=buf):
            iv = buf[pl.ds(j * 16, 16)]
            plsc.store_scatter(mask_v, [iv], ones16)

        if ci + 2 < NCH_C:
            descs[ci + 2] = pltpu.async_copy(
                idx_hbm.at[pl.ds(base + (ci + 2) * CHUNK, CHUNK)],
                buf, sems[ci % 2])

    if not combine:
        pltpu.sync_copy(mask_v, out_hbm.at[pl.ds(wid * M_PAD, M_PAD)])
        return

    hoff = s * HSLICE

    def pass_body(p, carry):
        plsc.subcore_barrier()
        pltpu.sync_copy(mask_v.at[pl.ds(p * HALF, HALF)],
                        shared.at[pl.ds(s * HALF, HALF)])
        plsc.subcore_barrier()

        pltpu.sync_copy(shared.at[pl.ds(hoff, HSLICE)],
                        mask_v.at[pl.ds(0, HSLICE)])

        def slot_body(j, carry2):
            pltpu.sync_copy(shared.at[pl.ds(j * HALF + hoff, HSLICE)],
                            mask_v.at[pl.ds(HSLICE, HSLICE)])

            @plsc.parallel_loop(0, HSLICE // 16, 1, unroll=4)
            def add_body(i):
                a = mask_v[pl.ds(i * 16, 16)]
                b = mask_v[pl.ds(HSLICE + i * 16, 16)]
                mask_v[pl.ds(i * 16, 16)] = a + b

            return carry2

        lax.fori_loop(1, NS, slot_body, None)
        pltpu.sync_copy(mask_v.at[pl.ds(0, HSLICE)],
                        out_hbm.at[pl.ds(c * M_PAD + p * HALF + hoff, HSLICE)])
        return carry

    lax.fori_loop(0, NPASS, pass_body, None)


_MESH = plsc.VectorSubcoreMesh(core_axis_name="c", subcore_axis_name="s")
_BASE_SCRATCH = [
    pltpu.VMEM((M_PAD,), jnp.float32),
    pltpu.VMEM((CHUNK,), jnp.int32),
    pltpu.VMEM((CHUNK,), jnp.int32),
]
_SEMS = [pltpu.SemaphoreType.DMA, pltpu.SemaphoreType.DMA]
_CP = pltpu.CompilerParams(needs_layout_passes=False)

_sc_first = functools.partial(
    pl.kernel, mesh=_MESH,
    out_type=jax.ShapeDtypeStruct((NW * M_PAD,), jnp.float32),
    scratch_types=_BASE_SCRATCH + _SEMS, compiler_params=_CP,
)(functools.partial(_sc_scatter, True, False))

_sc_mid = functools.partial(
    pl.kernel, mesh=_MESH,
    out_type=jax.ShapeDtypeStruct((NW * M_PAD,), jnp.float32),
    scratch_types=_BASE_SCRATCH + _SEMS, compiler_params=_CP,
)(functools.partial(_sc_scatter, False, False))

_sc_last = functools.partial(
    pl.kernel, mesh=_MESH,
    out_type=jax.ShapeDtypeStruct((NC * M_PAD,), jnp.float32),
    scratch_types=_BASE_SCRATCH
    + [pltpu.VMEM_SHARED((NS * HALF,), jnp.float32)] + _SEMS,
    compiler_params=_CP,
)(functools.partial(_sc_scatter, False, True))


def _tc_rowsum(t_ref, out_ref):
    tbt = t_ref[...].T
    r_t = jnp.sum(tbt, axis=0, keepdims=True)
    out_ref[...] = r_t.reshape(1, 1, ROW_BLK)


def _tc_rowsum_call(t):
    return pl.pallas_call(
        _tc_rowsum,
        grid=(NG,),
        in_specs=[
            pl.BlockSpec((ROW_BLK, N_COLS), lambda i: (i, 0)),
        ],
        out_specs=pl.BlockSpec((1, 1, ROW_BLK), lambda i: (i, 0, 0)),
        out_shape=jax.ShapeDtypeStruct((NG, 1, ROW_BLK), jnp.float32),
    )(t)


def _tc_final(hm_ref, r_ref, v_ref, out_ref):
    m = hm_ref[0] + hm_ref[1]
    r = r_ref[...]
    miss = jnp.where(m > 0.0, 0.0, r)
    nmiss = jnp.where(m > 0.0, 0.0, 1.0)
    s = jnp.sum(miss)
    n = jnp.sum(nmiss)
    v = v_ref[...]
    out_ref[...] = (s + (N_ROWS - n) * float(N_COLS) * v) / float(N_IDX)


def _tc_final_call(hm, rsum, val2d):
    return pl.pallas_call(
        _tc_final,
        grid=(1,),
        in_specs=[
            pl.BlockSpec((NC, NG, 1, ROW_BLK), lambda i: (0, 0, 0, 0)),
            pl.BlockSpec((NG, 1, ROW_BLK), lambda i: (0, 0, 0)),
            pl.BlockSpec((1, 1), lambda i: (0, 0)),
        ],
        out_specs=pl.BlockSpec((1, 1), lambda i: (0, 0)),
        out_shape=jax.ShapeDtypeStruct((1, 1), jnp.float32),
    )(hm, rsum, val2d)


def kernel(t):
    assert t.shape == (N_ROWS, N_COLS)
    k1, k2 = jax.random.split(jax.random.key(1))
    val = jax.random.normal(k2, (1,), dtype=t.dtype)

    _, klo = jax.random.split(k1)
    kd = jax.random.key_data(klo)
    span = jnp.uint32(SPAN)

    state = None
    hits = None
    for ci in range(NCHK):
        cnt = lax.iota(jnp.uint32, CH_N) + jnp.uint32(ci * CH_N)
        zero = jnp.zeros((CH_N,), jnp.uint32)
        b1, b2 = threefry2x32_p.bind(kd[0], kd[1], zero, cnt)
        idx_c = ((b1 ^ b2) % span).astype(jnp.int32)
        if ci == 0:
            state = _sc_first(idx_c)
        elif ci < NCHK - 1:
            state = _sc_mid(idx_c, state)
        else:
            hits = _sc_last(idx_c, state)

    hm = hits.reshape(NC, M_PAD)[:, :N_ROWS].reshape(NC, NG, 1, ROW_BLK)
    rsum = _tc_rowsum_call(t)
    out = _tc_final_call(hm, rsum, val.reshape(1, 1))
    return out[0, 0]


# speedup vs baseline: 1.0616x; 1.0454x over previous
import functools

import jax
import jax.numpy as jnp
from jax import lax
from jax.experimental import pallas as pl
from jax.experimental.pallas import tpu as pltpu
from jax.experimental.pallas import tpu_sc as plsc
from jax._src.random.threefry2x32 import threefry2x32_p

N_ROWS = 100000
N_COLS = 256
N_IDX = N_ROWS * N_COLS
NC, NS = 2, 16
NW = NC * NS
NCHK = 4
CH_N = N_IDX // NCHK
PER_C = CH_N // NW
CHUNK = 10000
NCH_C = PER_C // CHUNK
M_PAD = 100352
NPASS = 14
HALF = M_PAD // NPASS
HSLICE = HALF // NS
ROW_BLK = 2000
NG = N_ROWS // ROW_BLK
SPAN = 100000


def _sc_scatter(init, combine, idx_hbm, *refs):
    if init:
        (out_hbm, mask_v, buf0, buf1, sem0, sem1) = refs
        state_in = shared = None
    elif combine:
        (state_in, out_hbm, mask_v, buf0, buf1, shared, sem0, sem1) = refs
    else:
        (state_in, out_hbm, mask_v, buf0, buf1, sem0, sem1) = refs
        shared = None

    c = lax.axis_index("c")
    s = lax.axis_index("s")
    wid = s * NC + c
    base = wid * PER_C

    zeros16 = jnp.zeros((16,), jnp.float32)
    ones16 = jnp.ones((16,), jnp.float32)

    bufs = (buf0, buf1)
    sems = (sem0, sem1)
    descs = [None] * NCH_C
    for ci in range(2):
        descs[ci] = pltpu.async_copy(
            idx_hbm.at[pl.ds(base + ci * CHUNK, CHUNK)], bufs[ci], sems[ci])

    if init:
        @plsc.parallel_loop(0, M_PAD // 16, 1, unroll=8)
        def zero_body(i):
            mask_v[pl.ds(i * 16, 16)] = zeros16
    else:
        pltpu.sync_copy(state_in.at[pl.ds(wid * M_PAD, M_PAD)], mask_v)

    for ci in range(NCH_C):
        buf = bufs[ci % 2]
        descs[ci].wait()

        @plsc.parallel_loop(0, CHUNK // 16, 1, unroll=8)
        def scat_body(j, buf=buf):
            iv = buf[pl.ds(j * 16, 16)]
            plsc.store_scatter(mask_v, [iv], ones16)

        if ci + 2 < NCH_C:
            descs[ci + 2] = pltpu.async_copy(
                idx_hbm.at[pl.ds(base + (ci + 2) * CHUNK, CHUNK)],
                buf, sems[ci % 2])

    if not combine:
        pltpu.sync_copy(mask_v, out_hbm.at[pl.ds(wid * M_PAD, M_PAD)])
        return

    hoff = s * HSLICE

    def pass_body(p, carry):
        plsc.subcore_barrier()
        pltpu.sync_copy(mask_v.at[pl.ds(p * HALF, HALF)],
                        shared.at[pl.ds(s * HALF, HALF)])
        plsc.subcore_barrier()

        pltpu.sync_copy(shared.at[pl.ds(hoff, HSLICE)],
                        mask_v.at[pl.ds(0, HSLICE)])

        def slot_body(j, carry2):
            pltpu.sync_copy(shared.at[pl.ds(j * HALF + hoff, HSLICE)],
                            mask_v.at[pl.ds(HSLICE, HSLICE)])

            @plsc.parallel_loop(0, HSLICE // 16, 1, unroll=4)
            def add_body(i):
                a = mask_v[pl.ds(i * 16, 16)]
                b = mask_v[pl.ds(HSLICE + i * 16, 16)]
                mask_v[pl.ds(i * 16, 16)] = a + b

            return carry2

        lax.fori_loop(1, NS, slot_body, None)
        pltpu.sync_copy(mask_v.at[pl.ds(0, HSLICE)],
                        out_hbm.at[pl.ds(c * M_PAD + p * HALF + hoff, HSLICE)])
        return carry

    lax.fori_loop(0, NPASS, pass_body, None)


_MESH = plsc.VectorSubcoreMesh(core_axis_name="c", subcore_axis_name="s")
_BASE_SCRATCH = [
    pltpu.VMEM((M_PAD,), jnp.float32),
    pltpu.VMEM((CHUNK,), jnp.int32),
    pltpu.VMEM((CHUNK,), jnp.int32),
]
_SEMS = [pltpu.SemaphoreType.DMA, pltpu.SemaphoreType.DMA]
_CP = pltpu.CompilerParams(needs_layout_passes=False)

_sc_first = functools.partial(
    pl.kernel, mesh=_MESH,
    out_type=jax.ShapeDtypeStruct((NW * M_PAD,), jnp.float32),
    scratch_types=_BASE_SCRATCH + _SEMS, compiler_params=_CP,
)(functools.partial(_sc_scatter, True, False))

_sc_mid = functools.partial(
    pl.kernel, mesh=_MESH,
    out_type=jax.ShapeDtypeStruct((NW * M_PAD,), jnp.float32),
    scratch_types=_BASE_SCRATCH + _SEMS, compiler_params=_CP,
)(functools.partial(_sc_scatter, False, False))

_sc_last = functools.partial(
    pl.kernel, mesh=_MESH,
    out_type=jax.ShapeDtypeStruct((NC * M_PAD,), jnp.float32),
    scratch_types=_BASE_SCRATCH
    + [pltpu.VMEM_SHARED((NS * HALF,), jnp.float32)] + _SEMS,
    compiler_params=_CP,
)(functools.partial(_sc_scatter, False, True))


def _tc_rowsum(t_ref, out_ref):
    tbt = t_ref[...].T
    r_t = jnp.sum(tbt, axis=0, keepdims=True)
    out_ref[...] = r_t.reshape(1, 1, ROW_BLK)


def _tc_rowsum_call(t):
    return pl.pallas_call(
        _tc_rowsum,
        grid=(NG,),
        in_specs=[
            pl.BlockSpec((ROW_BLK, N_COLS), lambda i: (i, 0)),
        ],
        out_specs=pl.BlockSpec((1, 1, ROW_BLK), lambda i: (i, 0, 0)),
        out_shape=jax.ShapeDtypeStruct((NG, 1, ROW_BLK), jnp.float32),
    )(t)


def _tc_final(hm_ref, r_ref, v_ref, out_ref):
    m = hm_ref[0] + hm_ref[1]
    r = r_ref[...]
    miss = jnp.where(m > 0.0, 0.0, r)
    nmiss = jnp.where(m > 0.0, 0.0, 1.0)
    s = jnp.sum(miss)
    n = jnp.sum(nmiss)
    v = v_ref[...]
    out_ref[...] = (s + (N_ROWS - n) * float(N_COLS) * v) / float(N_IDX)


def _tc_final_call(hm, rsum, val2d):
    return pl.pallas_call(
        _tc_final,
        grid=(1,),
        in_specs=[
            pl.BlockSpec((NC, NG, 1, ROW_BLK), lambda i: (0, 0, 0, 0)),
            pl.BlockSpec((NG, 1, ROW_BLK), lambda i: (0, 0, 0)),
            pl.BlockSpec((1, 1), lambda i: (0, 0)),
        ],
        out_specs=pl.BlockSpec((1, 1), lambda i: (0, 0)),
        out_shape=jax.ShapeDtypeStruct((1, 1), jnp.float32),
    )(hm, rsum, val2d)


def kernel(t):
    assert t.shape == (N_ROWS, N_COLS)
    k1, k2 = jax.random.split(jax.random.key(1))
    val = jax.random.normal(k2, (1,), dtype=t.dtype)

    _, klo = jax.random.split(k1)
    kd = jax.random.key_data(klo)
    span = jnp.uint32(SPAN)

    state = None
    hits = None
    for ci in range(NCHK):
        cnt = lax.iota(jnp.uint32, CH_N) + jnp.uint32(ci * CH_N)
        zero = jnp.zeros((CH_N,), jnp.uint32)
        b1, b2 = threefry2x32_p.bind(kd[0], kd[1], zero, cnt)
        idx_c = ((b1 ^ b2) % span).astype(jnp.int32)
        if ci == 0:
            state = _sc_first(idx_c)
        elif ci < NCHK - 1:
            state = _sc_mid(idx_c, state)
        else:
            hits = _sc_last(idx_c, state)

    hm = hits.reshape(NC, M_PAD)[:, :N_ROWS].reshape(NC, NG, 1, ROW_BLK)
    rsum = _tc_rowsum_call(t)
    out = _tc_final_call(hm, rsum, val.reshape(1, 1))
    return out[0, 0]


# speedup vs baseline: 1.0698x; 1.0078x over previous
import functools

import jax
import jax.numpy as jnp
from jax import lax
from jax.experimental import pallas as pl
from jax.experimental.pallas import tpu as pltpu
from jax.experimental.pallas import tpu_sc as plsc
from jax._src.random.threefry2x32 import threefry2x32_p

N_ROWS = 100000
N_COLS = 256
N_IDX = N_ROWS * N_COLS
NC, NS = 2, 16
NW = NC * NS
NCHK = 4
CH_N = N_IDX // NCHK
PER_C = CH_N // NW
CHUNK = 10000
NCH_C = PER_C // CHUNK
M_PAD = 100352
NPASS = 14
HALF = M_PAD // NPASS
HSLICE = HALF // NS
ROW_BLK = 4000
NG = N_ROWS // ROW_BLK
SPAN = 100000


def _sc_scatter(init, combine, idx_hbm, *refs):
    if init:
        (out_hbm, mask_v, buf0, buf1, sem0, sem1) = refs
        state_in = shared = None
    elif combine:
        (state_in, out_hbm, mask_v, buf0, buf1, shared, sem0, sem1) = refs
    else:
        (state_in, out_hbm, mask_v, buf0, buf1, sem0, sem1) = refs
        shared = None

    c = lax.axis_index("c")
    s = lax.axis_index("s")
    wid = s * NC + c
    base = wid * PER_C

    zeros16 = jnp.zeros((16,), jnp.float32)
    ones16 = jnp.ones((16,), jnp.float32)

    bufs = (buf0, buf1)
    sems = (sem0, sem1)
    descs = [None] * NCH_C
    for ci in range(2):
        descs[ci] = pltpu.async_copy(
            idx_hbm.at[pl.ds(base + ci * CHUNK, CHUNK)], bufs[ci], sems[ci])

    if init:
        @plsc.parallel_loop(0, M_PAD // 16, 1, unroll=8)
        def zero_body(i):
            mask_v[pl.ds(i * 16, 16)] = zeros16
    else:
        pltpu.sync_copy(state_in.at[pl.ds(wid * M_PAD, M_PAD)], mask_v)

    for ci in range(NCH_C):
        buf = bufs[ci % 2]
        descs[ci].wait()

        @plsc.parallel_loop(0, CHUNK // 16, 1, unroll=8)
        def scat_body(j, buf=buf):
            iv = buf[pl.ds(j * 16, 16)]
            plsc.store_scatter(mask_v, [iv], ones16)

        if ci + 2 < NCH_C:
            descs[ci + 2] = pltpu.async_copy(
                idx_hbm.at[pl.ds(base + (ci + 2) * CHUNK, CHUNK)],
                buf, sems[ci % 2])

    if not combine:
        pltpu.sync_copy(mask_v, out_hbm.at[pl.ds(wid * M_PAD, M_PAD)])
        return

    hoff = s * HSLICE

    def pass_body(p, carry):
        plsc.subcore_barrier()
        pltpu.sync_copy(mask_v.at[pl.ds(p * HALF, HALF)],
                        shared.at[pl.ds(s * HALF, HALF)])
        plsc.subcore_barrier()

        pltpu.sync_copy(shared.at[pl.ds(hoff, HSLICE)],
                        mask_v.at[pl.ds(0, HSLICE)])

        def slot_body(j, carry2):
            pltpu.sync_copy(shared.at[pl.ds(j * HALF + hoff, HSLICE)],
                            mask_v.at[pl.ds(HSLICE, HSLICE)])

            @plsc.parallel_loop(0, HSLICE // 16, 1, unroll=4)
            def add_body(i):
                a = mask_v[pl.ds(i * 16, 16)]
                b = mask_v[pl.ds(HSLICE + i * 16, 16)]
                mask_v[pl.ds(i * 16, 16)] = a + b

            return carry2

        lax.fori_loop(1, NS, slot_body, None)
        pltpu.sync_copy(mask_v.at[pl.ds(0, HSLICE)],
                        out_hbm.at[pl.ds(c * M_PAD + p * HALF + hoff, HSLICE)])
        return carry

    lax.fori_loop(0, NPASS, pass_body, None)


_MESH = plsc.VectorSubcoreMesh(core_axis_name="c", subcore_axis_name="s")
_BASE_SCRATCH = [
    pltpu.VMEM((M_PAD,), jnp.float32),
    pltpu.VMEM((CHUNK,), jnp.int32),
    pltpu.VMEM((CHUNK,), jnp.int32),
]
_SEMS = [pltpu.SemaphoreType.DMA, pltpu.SemaphoreType.DMA]
_CP = pltpu.CompilerParams(needs_layout_passes=False)

_sc_first = functools.partial(
    pl.kernel, mesh=_MESH,
    out_type=jax.ShapeDtypeStruct((NW * M_PAD,), jnp.float32),
    scratch_types=_BASE_SCRATCH + _SEMS, compiler_params=_CP,
)(functools.partial(_sc_scatter, True, False))

_sc_mid = functools.partial(
    pl.kernel, mesh=_MESH,
    out_type=jax.ShapeDtypeStruct((NW * M_PAD,), jnp.float32),
    scratch_types=_BASE_SCRATCH + _SEMS, compiler_params=_CP,
)(functools.partial(_sc_scatter, False, False))

_sc_last = functools.partial(
    pl.kernel, mesh=_MESH,
    out_type=jax.ShapeDtypeStruct((NC * M_PAD,), jnp.float32),
    scratch_types=_BASE_SCRATCH
    + [pltpu.VMEM_SHARED((NS * HALF,), jnp.float32)] + _SEMS,
    compiler_params=_CP,
)(functools.partial(_sc_scatter, False, True))


def _tc_rowsum(t_ref, out_ref):
    tbt = t_ref[...].T
    r_t = jnp.sum(tbt, axis=0, keepdims=True)
    out_ref[...] = r_t.reshape(1, 1, ROW_BLK)


def _tc_rowsum_call(t):
    return pl.pallas_call(
        _tc_rowsum,
        grid=(NG,),
        in_specs=[
            pl.BlockSpec((ROW_BLK, N_COLS), lambda i: (i, 0)),
        ],
        out_specs=pl.BlockSpec((1, 1, ROW_BLK), lambda i: (i, 0, 0)),
        out_shape=jax.ShapeDtypeStruct((NG, 1, ROW_BLK), jnp.float32),
    )(t)


def _tc_final(hm_ref, r_ref, v_ref, out_ref):
    m = hm_ref[0] + hm_ref[1]
    r = r_ref[...]
    miss = jnp.where(m > 0.0, 0.0, r)
    nmiss = jnp.where(m > 0.0, 0.0, 1.0)
    s = jnp.sum(miss)
    n = jnp.sum(nmiss)
    v = v_ref[...]
    out_ref[...] = (s + (N_ROWS - n) * float(N_COLS) * v) / float(N_IDX)


def _tc_final_call(hm, rsum, val2d):
    return pl.pallas_call(
        _tc_final,
        grid=(1,),
        in_specs=[
            pl.BlockSpec((NC, NG, 1, ROW_BLK), lambda i: (0, 0, 0, 0)),
            pl.BlockSpec((NG, 1, ROW_BLK), lambda i: (0, 0, 0)),
            pl.BlockSpec((1, 1), lambda i: (0, 0)),
        ],
        out_specs=pl.BlockSpec((1, 1), lambda i: (0, 0)),
        out_shape=jax.ShapeDtypeStruct((1, 1), jnp.float32),
    )(hm, rsum, val2d)


def kernel(t):
    assert t.shape == (N_ROWS, N_COLS)
    k1, k2 = jax.random.split(jax.random.key(1))
    val = jax.random.normal(k2, (1,), dtype=t.dtype)

    _, klo = jax.random.split(k1)
    kd = jax.random.key_data(klo)
    span = jnp.uint32(SPAN)

    state = None
    hits = None
    for ci in range(NCHK):
        cnt = lax.iota(jnp.uint32, CH_N) + jnp.uint32(ci * CH_N)
        zero = jnp.zeros((CH_N,), jnp.uint32)
        b1, b2 = threefry2x32_p.bind(kd[0], kd[1], zero, cnt)
        idx_c = ((b1 ^ b2) % span).astype(jnp.int32)
        if ci == 0:
            state = _sc_first(idx_c)
        elif ci < NCHK - 1:
            state = _sc_mid(idx_c, state)
        else:
            hits = _sc_last(idx_c, state)

    hm = hits.reshape(NC, M_PAD)[:, :N_ROWS].reshape(NC, NG, 1, ROW_BLK)
    rsum = _tc_rowsum_call(t)
    out = _tc_final_call(hm, rsum, val.reshape(1, 1))
    return out[0, 0]


# speedup vs baseline: 1.0905x; 1.0193x over previous
import functools

import jax
import jax.numpy as jnp
from jax import lax
from jax.experimental import pallas as pl
from jax.experimental.pallas import tpu as pltpu
from jax.experimental.pallas import tpu_sc as plsc
from jax._src.random.threefry2x32 import threefry2x32_p

N_ROWS = 100000
N_COLS = 256
N_IDX = N_ROWS * N_COLS
NC, NS = 2, 16
NW = NC * NS
CHUNK = 10000
CH_SIZES = (7_680_000, 7_680_000, 7_680_000, 2_560_000)
NCHK = len(CH_SIZES)
assert sum(CH_SIZES) == N_IDX
M_PAD = 100352
NPASS = 14
HALF = M_PAD // NPASS
HSLICE = HALF // NS
ROW_BLK = 4000
NG = N_ROWS // ROW_BLK
SPAN = 100000


def _sc_scatter(init, combine, nch, idx_hbm, *refs):
    if init:
        (out_hbm, mask_v, buf0, buf1, sem0, sem1) = refs
        state_in = shared = None
    elif combine:
        (state_in, out_hbm, mask_v, buf0, buf1, shared, sem0, sem1) = refs
    else:
        (state_in, out_hbm, mask_v, buf0, buf1, sem0, sem1) = refs
        shared = None

    c = lax.axis_index("c")
    s = lax.axis_index("s")
    wid = s * NC + c
    base = wid * (nch * CHUNK)

    zeros16 = jnp.zeros((16,), jnp.float32)
    ones16 = jnp.ones((16,), jnp.float32)

    bufs = (buf0, buf1)
    sems = (sem0, sem1)
    descs = [None] * nch
    for ci in range(2):
        descs[ci] = pltpu.async_copy(
            idx_hbm.at[pl.ds(base + ci * CHUNK, CHUNK)], bufs[ci], sems[ci])

    if init:
        @plsc.parallel_loop(0, M_PAD // 16, 1, unroll=8)
        def zero_body(i):
            mask_v[pl.ds(i * 16, 16)] = zeros16
    else:
        pltpu.sync_copy(state_in.at[pl.ds(wid * M_PAD, M_PAD)], mask_v)

    for ci in range(nch):
        buf = bufs[ci % 2]
        descs[ci].wait()

        @plsc.parallel_loop(0, CHUNK // 16, 1, unroll=8)
        def scat_body(j, buf=buf):
            iv = buf[pl.ds(j * 16, 16)]
            plsc.store_scatter(mask_v, [iv], ones16)

        if ci + 2 < nch:
            descs[ci + 2] = pltpu.async_copy(
                idx_hbm.at[pl.ds(base + (ci + 2) * CHUNK, CHUNK)],
                buf, sems[ci % 2])

    if not combine:
        pltpu.sync_copy(mask_v, out_hbm.at[pl.ds(wid * M_PAD, M_PAD)])
        return

    hoff = s * HSLICE

    def pass_body(p, carry):
        plsc.subcore_barrier()
        pltpu.sync_copy(mask_v.at[pl.ds(p * HALF, HALF)],
                        shared.at[pl.ds(s * HALF, HALF)])
        plsc.subcore_barrier()

        pltpu.sync_copy(shared.at[pl.ds(hoff, HSLICE)],
                        mask_v.at[pl.ds(0, HSLICE)])

        def slot_body(j, carry2):
            pltpu.sync_copy(shared.at[pl.ds(j * HALF + hoff, HSLICE)],
                            mask_v.at[pl.ds(HSLICE, HSLICE)])

            @plsc.parallel_loop(0, HSLICE // 16, 1, unroll=4)
            def add_body(i):
                a = mask_v[pl.ds(i * 16, 16)]
                b = mask_v[pl.ds(HSLICE + i * 16, 16)]
                mask_v[pl.ds(i * 16, 16)] = a + b

            return carry2

        lax.fori_loop(1, NS, slot_body, None)
        pltpu.sync_copy(mask_v.at[pl.ds(0, HSLICE)],
                        out_hbm.at[pl.ds(c * M_PAD + p * HALF + hoff, HSLICE)])
        return carry

    lax.fori_loop(0, NPASS, pass_body, None)


_MESH = plsc.VectorSubcoreMesh(core_axis_name="c", subcore_axis_name="s")
_BASE_SCRATCH = [
    pltpu.VMEM((M_PAD,), jnp.float32),
    pltpu.VMEM((CHUNK,), jnp.int32),
    pltpu.VMEM((CHUNK,), jnp.int32),
]
_SEMS = [pltpu.SemaphoreType.DMA, pltpu.SemaphoreType.DMA]
_CP = pltpu.CompilerParams(needs_layout_passes=False)

_NCH_BIG = CH_SIZES[0] // NW // CHUNK
_NCH_LAST = CH_SIZES[-1] // NW // CHUNK

_sc_first = functools.partial(
    pl.kernel, mesh=_MESH,
    out_type=jax.ShapeDtypeStruct((NW * M_PAD,), jnp.float32),
    scratch_types=_BASE_SCRATCH + _SEMS, compiler_params=_CP,
)(functools.partial(_sc_scatter, True, False, _NCH_BIG))

_sc_mid = functools.partial(
    pl.kernel, mesh=_MESH,
    out_type=jax.ShapeDtypeStruct((NW * M_PAD,), jnp.float32),
    scratch_types=_BASE_SCRATCH + _SEMS, compiler_params=_CP,
)(functools.partial(_sc_scatter, False, False, _NCH_BIG))

_sc_last = functools.partial(
    pl.kernel, mesh=_MESH,
    out_type=jax.ShapeDtypeStruct((NC * M_PAD,), jnp.float32),
    scratch_types=_BASE_SCRATCH
    + [pltpu.VMEM_SHARED((NS * HALF,), jnp.float32)] + _SEMS,
    compiler_params=_CP,
)(functools.partial(_sc_scatter, False, True, _NCH_LAST))


def _tc_rowsum(t_ref, out_ref):
    tbt = t_ref[...].T
    r_t = jnp.sum(tbt, axis=0, keepdims=True)
    out_ref[...] = r_t.reshape(1, 1, ROW_BLK)


def _tc_rowsum_call(t):
    return pl.pallas_call(
        _tc_rowsum,
        grid=(NG,),
        in_specs=[
            pl.BlockSpec((ROW_BLK, N_COLS), lambda i: (i, 0)),
        ],
        out_specs=pl.BlockSpec((1, 1, ROW_BLK), lambda i: (i, 0, 0)),
        out_shape=jax.ShapeDtypeStruct((NG, 1, ROW_BLK), jnp.float32),
    )(t)


def _tc_final(hm_ref, r_ref, v_ref, out_ref):
    m = hm_ref[0] + hm_ref[1]
    r = r_ref[...]
    miss = jnp.where(m > 0.0, 0.0, r)
    nmiss = jnp.where(m > 0.0, 0.0, 1.0)
    s = jnp.sum(miss)
    n = jnp.sum(nmiss)
    v = v_ref[...]
    out_ref[...] = (s + (N_ROWS - n) * float(N_COLS) * v) / float(N_IDX)


def _tc_final_call(hm, rsum, val2d):
    return pl.pallas_call(
        _tc_final,
        grid=(1,),
        in_specs=[
            pl.BlockSpec((NC, NG, 1, ROW_BLK), lambda i: (0, 0, 0, 0)),
            pl.BlockSpec((NG, 1, ROW_BLK), lambda i: (0, 0, 0)),
            pl.BlockSpec((1, 1), lambda i: (0, 0)),
        ],
        out_specs=pl.BlockSpec((1, 1), lambda i: (0, 0)),
        out_shape=jax.ShapeDtypeStruct((1, 1), jnp.float32),
    )(hm, rsum, val2d)


def kernel(t):
    assert t.shape == (N_ROWS, N_COLS)
    k1, k2 = jax.random.split(jax.random.key(1))
    val = jax.random.normal(k2, (1,), dtype=t.dtype)

    _, klo = jax.random.split(k1)
    kd = jax.random.key_data(klo)
    span = jnp.uint32(SPAN)

    state = None
    hits = None
    off = 0
    for ci in range(NCHK):
        n = CH_SIZES[ci]
        cnt = lax.iota(jnp.uint32, n) + jnp.uint32(off)
        zero = jnp.zeros((n,), jnp.uint32)
        b1, b2 = threefry2x32_p.bind(kd[0], kd[1], zero, cnt)
        idx_c = ((b1 ^ b2) % span).astype(jnp.int32)
        if ci == 0:
            state = _sc_first(idx_c)
        elif ci < NCHK - 1:
            state = _sc_mid(idx_c, state)
        else:
            hits = _sc_last(idx_c, state)
        off += n

    hm = hits.reshape(NC, M_PAD)[:, :N_ROWS].reshape(NC, NG, 1, ROW_BLK)
    rsum = _tc_rowsum_call(t)
    out = _tc_final_call(hm, rsum, val.reshape(1, 1))
    return out[0, 0]
